# Initial kernel scaffold; baseline (speedup 1.0000x reference)
import jax
import jax.numpy as jnp
from jax.experimental import pallas as pl


def kernel(mask, attention):
    raise NotImplementedError("write your pallas kernel here")


# speedup vs baseline: 20.4213x; 20.4213x over previous
import functools

import jax
import jax.numpy as jnp
from jax import lax
from jax.experimental import pallas as pl
from jax.experimental.pallas import tpu as pltpu
from jax.experimental.pallas import tpu_sc as plsc

B, C, H, W = 8, 19, 512, 512
PIX = H * W
NC, NS = 2, 16
NW = NC * NS
PER_TILE = (B * PIX) // NW
CH = 2048
NCHUNK = PER_TILE // CH
NVREG = CH // 16


def _body(mask_hbm, att_hbm, outg_hbm, outl_hbm,
          att_v, g_v, l_v, in0, in1, og0, og1, ol0, ol1,
          sin0, sin1, sog0, sog1, sol0, sol1):
  wid = lax.axis_index("s") * NC + lax.axis_index("c")
  b = wid >> 2
  base = (wid & 3) * PER_TILE

  pltpu.sync_copy(att_hbm, att_v)
  for i in (0, 16):
    a0 = att_v[0, pl.ds(i, 16)]
    a1 = att_v[1, pl.ds(i, 16)]
    m = jnp.maximum(a0, a1)
    e0 = jnp.exp(a0 - m)
    e1 = jnp.exp(a1 - m)
    s = e0 + e1
    g_v[pl.ds(i, 16)] = e0 / s
    l_v[pl.ds(i, 16)] = e1 / s

  inbuf = (in0, in1)
  ogbuf = (og0, og1)
  olbuf = (ol0, ol1)
  isem = (sin0, sin1)
  gsem = (sog0, sog1)
  lsem = (sol0, sol1)

  def start_in(k):
    off = base + k * CH
    return pltpu.async_copy(
        mask_hbm.at[b, :, pl.ds(off, CH)], inbuf[k % 2], isem[k % 2])

  def compute(k):
    buf, og, ol = inbuf[k % 2], ogbuf[k % 2], olbuf[k % 2]

    def vbody(i, carry):
      s = i * 16
      best = buf[0, pl.ds(s, 16)]
      idx = jnp.zeros((16,), jnp.int32)
      for c in range(1, C):
        x = buf[c, pl.ds(s, 16)]
        upd = x > best
        best = jnp.where(upd, x, best)
        idx = jnp.where(upd, jnp.full((16,), c, jnp.int32), idx)
      og[pl.ds(s, 16)] = plsc.load_gather(g_v, [idx])
      ol[pl.ds(s, 16)] = plsc.load_gather(l_v, [idx])
      return carry

    lax.fori_loop(0, NVREG, vbody, 0)

  def start_out(k):
    off = base + k * CH
    hg = pltpu.async_copy(ogbuf[k % 2], outg_hbm.at[b, pl.ds(off, CH)],
                          gsem[k % 2])
    hl = pltpu.async_copy(olbuf[k % 2], outl_hbm.at[b, pl.ds(off, CH)],
                          lsem[k % 2])
    return hg, hl

  in_h = [None, None]
  out_h = [None, None]
  in_h[0] = start_in(0)
  for k in range(NCHUNK):
    p = k % 2
    if k + 1 < NCHUNK:
      in_h[1 - p] = start_in(k + 1)
    in_h[p].wait()
    if out_h[p] is not None:
      out_h[p][0].wait()
      out_h[p][1].wait()
    compute(k)
    out_h[p] = start_out(k)
  for p in range(2):
    out_h[p][0].wait()
    out_h[p][1].wait()


@jax.jit
def _run(mask3, att_pad):
  mesh = plsc.VectorSubcoreMesh(
      core_axis_name="c", subcore_axis_name="s",
      num_cores=NC, num_subcores=NS)
  f = pl.kernel(
      _body,
      out_type=(jax.ShapeDtypeStruct((B, PIX), jnp.float32),
                jax.ShapeDtypeStruct((B, PIX), jnp.float32)),
      mesh=mesh,
      scratch_types=[
          pltpu.VMEM((2, 32), jnp.float32),
          pltpu.VMEM((32,), jnp.float32),
          pltpu.VMEM((32,), jnp.float32),
          pltpu.VMEM((C, CH), jnp.float32),
          pltpu.VMEM((C, CH), jnp.float32),
          pltpu.VMEM((CH,), jnp.float32),
          pltpu.VMEM((CH,), jnp.float32),
          pltpu.VMEM((CH,), jnp.float32),
          pltpu.VMEM((CH,), jnp.float32),
          pltpu.SemaphoreType.DMA,
          pltpu.SemaphoreType.DMA,
          pltpu.SemaphoreType.DMA,
          pltpu.SemaphoreType.DMA,
          pltpu.SemaphoreType.DMA,
          pltpu.SemaphoreType.DMA,
      ],
      compiler_params=pltpu.CompilerParams(needs_layout_passes=False),
      name="attention_affine_sc",
  )
  return f(mask3, att_pad)


def kernel(mask, attention):
  mask3 = mask.reshape(B, C, PIX)
  att_pad = jnp.zeros((2, 32), jnp.float32).at[:, :C].set(attention.T)
  outg, outl = _run(mask3, att_pad)
  return (outg.reshape(B, 1, H, W), outl.reshape(B, 1, H, W))


# speedup vs baseline: 21.9631x; 1.0755x over previous
import functools

import jax
import jax.numpy as jnp
from jax import lax
from jax.experimental import pallas as pl
from jax.experimental.pallas import tpu as pltpu
from jax.experimental.pallas import tpu_sc as plsc

B, C, H, W = 8, 19, 512, 512
PIX = H * W
NC, NS = 2, 16
NW = NC * NS
PER_TILE = (B * PIX) // NW
CH = 2048
NCHUNK = PER_TILE // CH
NVREG = CH // 16


def _body(mask_hbm, att_hbm, outg_hbm, outl_hbm,
          att_v, g_v, l_v, in0, in1, og0, og1, ol0, ol1,
          sin0, sin1, sog0, sog1, sol0, sol1):
  wid = lax.axis_index("s") * NC + lax.axis_index("c")
  b = wid >> 2
  base = (wid & 3) * PER_TILE

  pltpu.sync_copy(att_hbm, att_v)
  for i in (0, 16):
    a0 = att_v[0, pl.ds(i, 16)]
    a1 = att_v[1, pl.ds(i, 16)]
    m = jnp.maximum(a0, a1)
    e0 = jnp.exp(a0 - m)
    e1 = jnp.exp(a1 - m)
    s = e0 + e1
    g_v[pl.ds(i, 16)] = e0 / s
    l_v[pl.ds(i, 16)] = e1 / s

  inbuf = (in0, in1)
  ogbuf = (og0, og1)
  olbuf = (ol0, ol1)
  isem = (sin0, sin1)
  gsem = (sog0, sog1)
  lsem = (sol0, sol1)

  def start_in(k):
    off = base + k * CH
    return pltpu.async_copy(
        mask_hbm.at[b, :, pl.ds(off, CH)], inbuf[k % 2], isem[k % 2])

  def compute(k):
    buf, og, ol = inbuf[k % 2], ogbuf[k % 2], olbuf[k % 2]

    def vbody(i, carry):
      s = i * 16

      def tree(lo, hi):
        if hi - lo == 1:
          return buf[lo, pl.ds(s, 16)], jnp.full((16,), lo, jnp.int32)
        mid = (lo + hi) // 2
        va, ia = tree(lo, mid)
        vb, ib = tree(mid, hi)
        upd = vb > va
        return jnp.where(upd, vb, va), jnp.where(upd, ib, ia)

      _, idx = tree(0, C)
      og[pl.ds(s, 16)] = plsc.load_gather(g_v, [idx])
      ol[pl.ds(s, 16)] = plsc.load_gather(l_v, [idx])
      return carry

    lax.fori_loop(0, NVREG, vbody, 0)

  def start_out(k):
    off = base + k * CH
    hg = pltpu.async_copy(ogbuf[k % 2], outg_hbm.at[b, pl.ds(off, CH)],
                          gsem[k % 2])
    hl = pltpu.async_copy(olbuf[k % 2], outl_hbm.at[b, pl.ds(off, CH)],
                          lsem[k % 2])
    return hg, hl

  in_h = [None, None]
  out_h = [None, None]
  in_h[0] = start_in(0)
  for k in range(NCHUNK):
    p = k % 2
    if k + 1 < NCHUNK:
      in_h[1 - p] = start_in(k + 1)
    in_h[p].wait()
    if out_h[p] is not None:
      out_h[p][0].wait()
      out_h[p][1].wait()
    compute(k)
    out_h[p] = start_out(k)
  for p in range(2):
    out_h[p][0].wait()
    out_h[p][1].wait()


@jax.jit
def _run(mask3, att_pad):
  mesh = plsc.VectorSubcoreMesh(
      core_axis_name="c", subcore_axis_name="s",
      num_cores=NC, num_subcores=NS)
  f = pl.kernel(
      _body,
      out_type=(jax.ShapeDtypeStruct((B, PIX), jnp.float32),
                jax.ShapeDtypeStruct((B, PIX), jnp.float32)),
      mesh=mesh,
      scratch_types=[
          pltpu.VMEM((2, 32), jnp.float32),
          pltpu.VMEM((32,), jnp.float32),
          pltpu.VMEM((32,), jnp.float32),
          pltpu.VMEM((C, CH), jnp.float32),
          pltpu.VMEM((C, CH), jnp.float32),
          pltpu.VMEM((CH,), jnp.float32),
          pltpu.VMEM((CH,), jnp.float32),
          pltpu.VMEM((CH,), jnp.float32),
          pltpu.VMEM((CH,), jnp.float32),
          pltpu.SemaphoreType.DMA,
          pltpu.SemaphoreType.DMA,
          pltpu.SemaphoreType.DMA,
          pltpu.SemaphoreType.DMA,
          pltpu.SemaphoreType.DMA,
          pltpu.SemaphoreType.DMA,
      ],
      compiler_params=pltpu.CompilerParams(needs_layout_passes=False),
      name="attention_affine_sc",
  )
  return f(mask3, att_pad)


def kernel(mask, attention):
  mask3 = mask.reshape(B, C, PIX)
  att_pad = jnp.zeros((2, 32), jnp.float32).at[:, :C].set(attention.T)
  outg, outl = _run(mask3, att_pad)
  return (outg.reshape(B, 1, H, W), outl.reshape(B, 1, H, W))


# speedup vs baseline: 25.2610x; 1.1502x over previous
import functools

import jax
import jax.numpy as jnp
from jax import lax
from jax.experimental import pallas as pl
from jax.experimental.pallas import tpu as pltpu
from jax.experimental.pallas import tpu_sc as plsc

B, C, H, W = 8, 19, 512, 512
PIX = H * W
NC, NS = 2, 16
NW = NC * NS
PER_TILE = (B * PIX) // NW
CH = 2048
NCHUNK = PER_TILE // CH
NVREG = CH // 16


def _body(mask_hbm, att_hbm, outg_hbm, outl_hbm,
          att_v, g_v, l_v, in0, in1, og0, og1, ol0, ol1,
          sin0, sin1, sog0, sog1, sol0, sol1):
  wid = lax.axis_index("s") * NC + lax.axis_index("c")
  b = wid >> 2
  base = (wid & 3) * PER_TILE

  pltpu.sync_copy(att_hbm, att_v)
  for i in (0, 16):
    a0 = att_v[0, pl.ds(i, 16)]
    a1 = att_v[1, pl.ds(i, 16)]
    m = jnp.maximum(a0, a1)
    e0 = jnp.exp(a0 - m)
    e1 = jnp.exp(a1 - m)
    s = e0 + e1
    g_v[pl.ds(i, 16)] = e0 / s
    l_v[pl.ds(i, 16)] = e1 / s

  inbuf = (in0, in1)
  ogbuf = (og0, og1)
  olbuf = (ol0, ol1)
  isem = (sin0, sin1)
  gsem = (sog0, sog1)
  lsem = (sol0, sol1)

  def start_in(k, p):
    off = base + k * CH
    pltpu.async_copy(mask_hbm.at[b, :, pl.ds(off, CH)], inbuf[p], isem[p])

  def wait_in(p):
    pltpu.make_async_copy(
        mask_hbm.at[b, :, pl.ds(base, CH)], inbuf[p], isem[p]).wait()

  def compute(k, p):
    off = base + k * CH
    buf, og, ol = inbuf[p], ogbuf[p], olbuf[p]

    @plsc.parallel_loop(0, NVREG, 1, unroll=4)
    def vbody(i):
      s = i * 16

      def tree(lo, hi):
        if hi - lo == 1:
          return buf[lo, pl.ds(s, 16)], jnp.full((16,), lo, jnp.int32)
        mid = (lo + hi) // 2
        va, ia = tree(lo, mid)
        vb, ib = tree(mid, hi)
        upd = vb > va
        return jnp.where(upd, vb, va), jnp.where(upd, ib, ia)

      _, idx = tree(0, C)
      og[pl.ds(s, 16)] = plsc.load_gather(g_v, [idx])
      ol[pl.ds(s, 16)] = plsc.load_gather(l_v, [idx])

    pltpu.async_copy(og, outg_hbm.at[b, pl.ds(off, CH)], gsem[p])
    pltpu.async_copy(ol, outl_hbm.at[b, pl.ds(off, CH)], lsem[p])

  def wait_out(p):
    pltpu.make_async_copy(
        ogbuf[p], outg_hbm.at[b, pl.ds(base, CH)], gsem[p]).wait()
    pltpu.make_async_copy(
        olbuf[p], outl_hbm.at[b, pl.ds(base, CH)], lsem[p]).wait()

  start_in(0, 0)

  def pair(j, carry):
    k0 = j * 2
    start_in(k0 + 1, 1)
    wait_in(0)
    lax.cond(j >= 1, lambda: wait_out(0), lambda: None)
    compute(k0, 0)

    lax.cond(j < NCHUNK // 2 - 1, lambda: start_in(k0 + 2, 0), lambda: None)
    wait_in(1)
    lax.cond(j >= 1, lambda: wait_out(1), lambda: None)
    compute(k0 + 1, 1)
    return carry

  lax.fori_loop(0, NCHUNK // 2, pair, 0)
  wait_out(0)
  wait_out(1)


@jax.jit
def _run(mask3, att_pad):
  mesh = plsc.VectorSubcoreMesh(
      core_axis_name="c", subcore_axis_name="s",
      num_cores=NC, num_subcores=NS)
  f = pl.kernel(
      _body,
      out_type=(jax.ShapeDtypeStruct((B, PIX), jnp.float32),
                jax.ShapeDtypeStruct((B, PIX), jnp.float32)),
      mesh=mesh,
      scratch_types=[
          pltpu.VMEM((2, 32), jnp.float32),
          pltpu.VMEM((32,), jnp.float32),
          pltpu.VMEM((32,), jnp.float32),
          pltpu.VMEM((C, CH), jnp.float32),
          pltpu.VMEM((C, CH), jnp.float32),
          pltpu.VMEM((CH,), jnp.float32),
          pltpu.VMEM((CH,), jnp.float32),
          pltpu.VMEM((CH,), jnp.float32),
          pltpu.VMEM((CH,), jnp.float32),
          pltpu.SemaphoreType.DMA,
          pltpu.SemaphoreType.DMA,
          pltpu.SemaphoreType.DMA,
          pltpu.SemaphoreType.DMA,
          pltpu.SemaphoreType.DMA,
          pltpu.SemaphoreType.DMA,
      ],
      compiler_params=pltpu.CompilerParams(needs_layout_passes=False),
      name="attention_affine_sc",
  )
  return f(mask3, att_pad)


def kernel(mask, attention):
  mask3 = mask.reshape(B, C, PIX)
  att_pad = jnp.zeros((2, 32), jnp.float32).at[:, :C].set(attention.T)
  outg, outl = _run(mask3, att_pad)
  return (outg.reshape(B, 1, H, W), outl.reshape(B, 1, H, W))


# speedup vs baseline: 84.0313x; 3.3265x over previous
import jax
import jax.numpy as jnp
from jax import lax
from jax.experimental import pallas as pl
from jax.experimental.pallas import tpu as pltpu
from jax.experimental.pallas import tpu_sc as plsc

B, C, H, W = 8, 19, 512, 512
NC, NS = 2, 16
NW = NC * NS
ROWS_PER_TILE = (B * H) // NW
RB = 4
CH = RB * W
NCHUNK = ROWS_PER_TILE // RB
NVREG = CH // 16
VPR = W // 16


def _body(mask_hbm, att_hbm, outg_hbm, outl_hbm,
          att_v, g_v, l_v, in0, in1, og0, og1, ol0, ol1,
          sin0, sin1, sog0, sog1, sol0, sol1):
  wid = lax.axis_index("s") * NC + lax.axis_index("c")
  b = wid >> 2
  row0 = (wid & 3) * ROWS_PER_TILE

  pltpu.sync_copy(att_hbm, att_v)
  for i in (0, 16):
    a0 = att_v[0, pl.ds(i, 16)]
    a1 = att_v[1, pl.ds(i, 16)]
    m = jnp.maximum(a0, a1)
    e0 = jnp.exp(a0 - m)
    e1 = jnp.exp(a1 - m)
    s = e0 + e1
    g_v[pl.ds(i, 16)] = e0 / s
    l_v[pl.ds(i, 16)] = e1 / s

  inbuf = (in0, in1)
  ogbuf = (og0, og1)
  olbuf = (ol0, ol1)
  isem = (sin0, sin1)
  gsem = (sog0, sog1)
  lsem = (sol0, sol1)

  def start_in(k, p):
    r = row0 + k * RB
    pltpu.async_copy(mask_hbm.at[b, :, pl.ds(r, RB), :], inbuf[p], isem[p])

  def wait_in(p):
    pltpu.make_async_copy(
        mask_hbm.at[b, :, pl.ds(row0, RB), :], inbuf[p], isem[p]).wait()

  def compute(k, p):
    r0 = row0 + k * RB
    buf, og, ol = inbuf[p], ogbuf[p], olbuf[p]

    @plsc.parallel_loop(0, NVREG, 1, unroll=4)
    def vbody(i):
      r = i // VPR
      col = (i % VPR) * 16

      def tree(lo, hi):
        if hi - lo == 1:
          return buf[lo, r, pl.ds(col, 16)], jnp.full((16,), lo, jnp.int32)
        mid = (lo + hi) // 2
        va, ia = tree(lo, mid)
        vb, ib = tree(mid, hi)
        upd = vb > va
        return jnp.where(upd, vb, va), jnp.where(upd, ib, ia)

      _, idx = tree(0, C)
      og[r, pl.ds(col, 16)] = plsc.load_gather(g_v, [idx])
      ol[r, pl.ds(col, 16)] = plsc.load_gather(l_v, [idx])

    pltpu.async_copy(og, outg_hbm.at[b, 0, pl.ds(r0, RB), :], gsem[p])
    pltpu.async_copy(ol, outl_hbm.at[b, 0, pl.ds(r0, RB), :], lsem[p])

  def wait_out(p):
    pltpu.make_async_copy(
        ogbuf[p], outg_hbm.at[b, 0, pl.ds(row0, RB), :], gsem[p]).wait()
    pltpu.make_async_copy(
        olbuf[p], outl_hbm.at[b, 0, pl.ds(row0, RB), :], lsem[p]).wait()

  start_in(0, 0)

  def pair(j, carry):
    k0 = j * 2
    start_in(k0 + 1, 1)
    wait_in(0)
    lax.cond(j >= 1, lambda: wait_out(0), lambda: None)
    compute(k0, 0)

    lax.cond(j < NCHUNK // 2 - 1, lambda: start_in(k0 + 2, 0), lambda: None)
    wait_in(1)
    lax.cond(j >= 1, lambda: wait_out(1), lambda: None)
    compute(k0 + 1, 1)
    return carry

  lax.fori_loop(0, NCHUNK // 2, pair, 0)
  wait_out(0)
  wait_out(1)


@jax.jit
def _run(mask, att_pad):
  mesh = plsc.VectorSubcoreMesh(
      core_axis_name="c", subcore_axis_name="s",
      num_cores=NC, num_subcores=NS)
  f = pl.kernel(
      _body,
      out_type=(jax.ShapeDtypeStruct((B, 1, H, W), jnp.float32),
                jax.ShapeDtypeStruct((B, 1, H, W), jnp.float32)),
      mesh=mesh,
      scratch_types=[
          pltpu.VMEM((2, 32), jnp.float32),
          pltpu.VMEM((32,), jnp.float32),
          pltpu.VMEM((32,), jnp.float32),
          pltpu.VMEM((C, RB, W), jnp.float32),
          pltpu.VMEM((C, RB, W), jnp.float32),
          pltpu.VMEM((RB, W), jnp.float32),
          pltpu.VMEM((RB, W), jnp.float32),
          pltpu.VMEM((RB, W), jnp.float32),
          pltpu.VMEM((RB, W), jnp.float32),
          pltpu.SemaphoreType.DMA,
          pltpu.SemaphoreType.DMA,
          pltpu.SemaphoreType.DMA,
          pltpu.SemaphoreType.DMA,
          pltpu.SemaphoreType.DMA,
          pltpu.SemaphoreType.DMA,
      ],
      compiler_params=pltpu.CompilerParams(needs_layout_passes=False),
      name="attention_affine_sc",
  )
  return f(mask, att_pad)


def kernel(mask, attention):
  att_pad = jnp.zeros((2, 32), jnp.float32).at[:, :C].set(attention.T)
  return _run(mask, att_pad)


# speedup vs baseline: 89.3684x; 1.0635x over previous
import jax
import jax.numpy as jnp
from jax import lax
from jax.experimental import pallas as pl
from jax.experimental.pallas import tpu as pltpu
from jax.experimental.pallas import tpu_sc as plsc

B, C, H, W = 8, 19, 512, 512
NC, NS = 2, 16
NW = NC * NS
SC_B = 4
TC_B = B - SC_B
RB = 8
WC = 256
CH = RB * WC
ROWS_PER_TILE = (SC_B * H) // NW
NWH = W // WC
NCHUNK = (ROWS_PER_TILE // RB) * NWH
NVREG = CH // 16
VPR = WC // 16
HB = 64


def _sc_body(mask_hbm, att_hbm, outg_hbm, outl_hbm,
             att_v, g_v, l_v, in0, in1, og0, og1, ol0, ol1,
             sin0, sin1, sog0, sog1, sol0, sol1):
  wid = lax.axis_index("s") * NC + lax.axis_index("c")
  row_base = wid * ROWS_PER_TILE

  pltpu.sync_copy(att_hbm, att_v)
  for i in (0, 16):
    a0 = att_v[0, pl.ds(i, 16)]
    a1 = att_v[1, pl.ds(i, 16)]
    m = jnp.maximum(a0, a1)
    e0 = jnp.exp(a0 - m)
    e1 = jnp.exp(a1 - m)
    s = e0 + e1
    g_v[pl.ds(i, 16)] = e0 / s
    l_v[pl.ds(i, 16)] = e1 / s

  inbuf = (in0, in1)
  ogbuf = (og0, og1)
  olbuf = (ol0, ol1)
  isem = (sin0, sin1)
  gsem = (sog0, sog1)
  lsem = (sol0, sol1)

  def chunk_addr(k):
    g = row_base + (k // NWH) * RB
    wcol = (k % NWH) * WC
    return g >> 9, pl.multiple_of(g & (H - 1), RB), pl.multiple_of(wcol, WC)

  def start_in(k, p):
    b, r, wcol = chunk_addr(k)
    pltpu.async_copy(
        mask_hbm.at[b, :, pl.ds(r, RB), pl.ds(wcol, WC)], inbuf[p], isem[p])

  def wait_in(p):
    pltpu.make_async_copy(
        mask_hbm.at[0, :, pl.ds(0, RB), pl.ds(0, WC)], inbuf[p], isem[p]).wait()

  def compute(k, p):
    b, r0, wcol = chunk_addr(k)
    buf, og, ol = inbuf[p], ogbuf[p], olbuf[p]

    @plsc.parallel_loop(0, NVREG, 1, unroll=4)
    def vbody(i):
      r = i // VPR
      col = (i % VPR) * 16

      def tree(lo, hi):
        if hi - lo == 1:
          return buf[lo, r, pl.ds(col, 16)], jnp.full((16,), lo, jnp.int32)
        mid = (lo + hi) // 2
        va, ia = tree(lo, mid)
        vb, ib = tree(mid, hi)
        upd = vb > va
        return jnp.where(upd, vb, va), jnp.where(upd, ib, ia)

      _, idx = tree(0, C)
      og[r, pl.ds(col, 16)] = plsc.load_gather(g_v, [idx])
      ol[r, pl.ds(col, 16)] = plsc.load_gather(l_v, [idx])

    pltpu.async_copy(
        og, outg_hbm.at[b, 0, pl.ds(r0, RB), pl.ds(wcol, WC)], gsem[p])
    pltpu.async_copy(
        ol, outl_hbm.at[b, 0, pl.ds(r0, RB), pl.ds(wcol, WC)], lsem[p])

  def wait_out(p):
    pltpu.make_async_copy(
        ogbuf[p], outg_hbm.at[0, 0, pl.ds(0, RB), pl.ds(0, WC)], gsem[p]).wait()
    pltpu.make_async_copy(
        olbuf[p], outl_hbm.at[0, 0, pl.ds(0, RB), pl.ds(0, WC)], lsem[p]).wait()

  start_in(0, 0)

  def pair(j, carry):
    k0 = j * 2
    start_in(k0 + 1, 1)
    wait_in(0)
    lax.cond(j >= 1, lambda: wait_out(0), lambda: None)
    compute(k0, 0)

    lax.cond(j < NCHUNK // 2 - 1, lambda: start_in(k0 + 2, 0), lambda: None)
    wait_in(1)
    lax.cond(j >= 1, lambda: wait_out(1), lambda: None)
    compute(k0 + 1, 1)
    return carry

  lax.fori_loop(0, NCHUNK // 2, pair, 0)
  wait_out(0)
  wait_out(1)


def _tc_body(att_ref, mask_ref, og_ref, ol_ref):
  best = mask_ref[0, 0]
  bd = jnp.full((HB, W), att_ref[0, 0] - att_ref[0, 1], jnp.float32)
  for c in range(1, C):
    x = mask_ref[0, c]
    upd = x > best
    best = jnp.where(upd, x, best)
    bd = jnp.where(upd, att_ref[c, 0] - att_ref[c, 1], bd)
  og_ref[0, 0] = 1.0 / (1.0 + jnp.exp(-bd))
  ol_ref[0, 0] = 1.0 / (1.0 + jnp.exp(bd))


@jax.jit
def _run(mask, attention, att_pad):
  mesh = plsc.VectorSubcoreMesh(
      core_axis_name="c", subcore_axis_name="s",
      num_cores=NC, num_subcores=NS)
  sc = pl.kernel(
      _sc_body,
      out_type=(jax.ShapeDtypeStruct((SC_B, 1, H, W), jnp.float32),
                jax.ShapeDtypeStruct((SC_B, 1, H, W), jnp.float32)),
      mesh=mesh,
      scratch_types=[
          pltpu.VMEM((2, 32), jnp.float32),
          pltpu.VMEM((32,), jnp.float32),
          pltpu.VMEM((32,), jnp.float32),
          pltpu.VMEM((C, RB, WC), jnp.float32),
          pltpu.VMEM((C, RB, WC), jnp.float32),
          pltpu.VMEM((RB, WC), jnp.float32),
          pltpu.VMEM((RB, WC), jnp.float32),
          pltpu.VMEM((RB, WC), jnp.float32),
          pltpu.VMEM((RB, WC), jnp.float32),
          pltpu.SemaphoreType.DMA,
          pltpu.SemaphoreType.DMA,
          pltpu.SemaphoreType.DMA,
          pltpu.SemaphoreType.DMA,
          pltpu.SemaphoreType.DMA,
          pltpu.SemaphoreType.DMA,
      ],
      compiler_params=pltpu.CompilerParams(needs_layout_passes=False),
      name="attention_affine_sc",
  )
  sc_g, sc_l = sc(mask, att_pad)

  tc = pl.pallas_call(
      _tc_body,
      grid=(TC_B, H // HB),
      in_specs=[
          pl.BlockSpec(memory_space=pltpu.SMEM),
          pl.BlockSpec((1, C, HB, W), lambda b, h: (SC_B + b, 0, h, 0)),
      ],
      out_specs=[
          pl.BlockSpec((1, 1, HB, W), lambda b, h: (b, 0, h, 0)),
          pl.BlockSpec((1, 1, HB, W), lambda b, h: (b, 0, h, 0)),
      ],
      out_shape=(jax.ShapeDtypeStruct((TC_B, 1, H, W), jnp.float32),
                 jax.ShapeDtypeStruct((TC_B, 1, H, W), jnp.float32)),
      name="attention_affine_tc",
  )
  tc_g, tc_l = tc(attention, mask)

  return (jnp.concatenate([sc_g, tc_g], axis=0),
          jnp.concatenate([sc_l, tc_l], axis=0))


def kernel(mask, attention):
  att_pad = jnp.zeros((2, 32), jnp.float32).at[:, :C].set(attention.T)
  return _run(mask, attention, att_pad)


# speedup vs baseline: 95.5942x; 1.0697x over previous
import jax
import jax.numpy as jnp
from jax import lax
from jax.experimental import pallas as pl
from jax.experimental.pallas import tpu as pltpu
from jax.experimental.pallas import tpu_sc as plsc

B, C, H, W = 8, 19, 512, 512
NC, NS = 2, 16
NW = NC * NS
SC_B = 4
TC_B = B - SC_B
RB = 8
WC = 256
CH = RB * WC
ROWS_PER_TILE = (SC_B * H) // NW
NWH = W // WC
NCHUNK = (ROWS_PER_TILE // RB) * NWH
NVREG = CH // 16
VPR = WC // 16
HB = 128


def _sc_body(mask_hbm, att_hbm, outg_hbm, outl_hbm,
             att_v, g_v, l_v, in0, in1, og0, og1, ol0, ol1,
             sin0, sin1, sog0, sog1, sol0, sol1):
  wid = lax.axis_index("s") * NC + lax.axis_index("c")
  row_base = wid * ROWS_PER_TILE

  pltpu.sync_copy(att_hbm, att_v)
  for i in (0, 16):
    rows = jnp.minimum(lax.iota(jnp.int32, 16) + i, C - 1)
    a0 = plsc.load_gather(att_v, [rows, jnp.zeros((16,), jnp.int32)])
    a1 = plsc.load_gather(att_v, [rows, jnp.ones((16,), jnp.int32)])
    m = jnp.maximum(a0, a1)
    e0 = jnp.exp(a0 - m)
    e1 = jnp.exp(a1 - m)
    s = e0 + e1
    g_v[pl.ds(i, 16)] = e0 / s
    l_v[pl.ds(i, 16)] = e1 / s

  inbuf = (in0, in1)
  ogbuf = (og0, og1)
  olbuf = (ol0, ol1)
  isem = (sin0, sin1)
  gsem = (sog0, sog1)
  lsem = (sol0, sol1)

  def chunk_addr(k):
    g = row_base + (k // NWH) * RB
    wcol = (k % NWH) * WC
    return g >> 9, pl.multiple_of(g & (H - 1), RB), pl.multiple_of(wcol, WC)

  def start_in(k, p):
    b, r, wcol = chunk_addr(k)
    pltpu.async_copy(
        mask_hbm.at[b, :, pl.ds(r, RB), pl.ds(wcol, WC)], inbuf[p], isem[p])

  def wait_in(p):
    pltpu.make_async_copy(
        mask_hbm.at[0, :, pl.ds(0, RB), pl.ds(0, WC)], inbuf[p], isem[p]).wait()

  def compute(k, p):
    b, r0, wcol = chunk_addr(k)
    buf, og, ol = inbuf[p], ogbuf[p], olbuf[p]

    @plsc.parallel_loop(0, NVREG, 1, unroll=2)
    def vbody(i):
      r = i // VPR
      col = (i % VPR) * 16

      def tree(lo, hi):
        if hi - lo == 1:
          return buf[lo, r, pl.ds(col, 16)], jnp.full((16,), lo, jnp.int32)
        mid = (lo + hi) // 2
        va, ia = tree(lo, mid)
        vb, ib = tree(mid, hi)
        upd = vb > va
        return jnp.where(upd, vb, va), jnp.where(upd, ib, ia)

      _, idx = tree(0, C)
      og[r, pl.ds(col, 16)] = plsc.load_gather(g_v, [idx])
      ol[r, pl.ds(col, 16)] = plsc.load_gather(l_v, [idx])

    pltpu.async_copy(
        og, outg_hbm.at[b, 0, pl.ds(r0, RB), pl.ds(wcol, WC)], gsem[p])
    pltpu.async_copy(
        ol, outl_hbm.at[b, 0, pl.ds(r0, RB), pl.ds(wcol, WC)], lsem[p])

  def wait_out(p):
    pltpu.make_async_copy(
        ogbuf[p], outg_hbm.at[0, 0, pl.ds(0, RB), pl.ds(0, WC)], gsem[p]).wait()
    pltpu.make_async_copy(
        olbuf[p], outl_hbm.at[0, 0, pl.ds(0, RB), pl.ds(0, WC)], lsem[p]).wait()

  start_in(0, 0)

  def pair(j, carry):
    k0 = j * 2
    start_in(k0 + 1, 1)
    wait_in(0)
    lax.cond(j >= 1, lambda: wait_out(0), lambda: None)
    compute(k0, 0)

    lax.cond(j < NCHUNK // 2 - 1, lambda: start_in(k0 + 2, 0), lambda: None)
    wait_in(1)
    lax.cond(j >= 1, lambda: wait_out(1), lambda: None)
    compute(k0 + 1, 1)
    return carry

  lax.fori_loop(0, NCHUNK // 2, pair, 0)
  wait_out(0)
  wait_out(1)


def _tc_body(att_ref, mask_ref, og_ref, ol_ref):
  best = mask_ref[0, 0]
  bd = jnp.full((HB, W), att_ref[0, 0] - att_ref[0, 1], jnp.float32)
  for c in range(1, C):
    x = mask_ref[0, c]
    upd = x > best
    best = jnp.where(upd, x, best)
    bd = jnp.where(upd, att_ref[c, 0] - att_ref[c, 1], bd)
  og_ref[0, 0] = 1.0 / (1.0 + jnp.exp(-bd))
  ol_ref[0, 0] = 1.0 / (1.0 + jnp.exp(bd))


@jax.jit
def _run(mask, attention):
  mesh = plsc.VectorSubcoreMesh(
      core_axis_name="c", subcore_axis_name="s",
      num_cores=NC, num_subcores=NS)
  sc = pl.kernel(
      _sc_body,
      out_type=(jax.ShapeDtypeStruct((B, 1, H, W), jnp.float32),
                jax.ShapeDtypeStruct((B, 1, H, W), jnp.float32)),
      mesh=mesh,
      scratch_types=[
          pltpu.VMEM((C, 2), jnp.float32),
          pltpu.VMEM((32,), jnp.float32),
          pltpu.VMEM((32,), jnp.float32),
          pltpu.VMEM((C, RB, WC), jnp.float32),
          pltpu.VMEM((C, RB, WC), jnp.float32),
          pltpu.VMEM((RB, WC), jnp.float32),
          pltpu.VMEM((RB, WC), jnp.float32),
          pltpu.VMEM((RB, WC), jnp.float32),
          pltpu.VMEM((RB, WC), jnp.float32),
          pltpu.SemaphoreType.DMA,
          pltpu.SemaphoreType.DMA,
          pltpu.SemaphoreType.DMA,
          pltpu.SemaphoreType.DMA,
          pltpu.SemaphoreType.DMA,
          pltpu.SemaphoreType.DMA,
      ],
      compiler_params=pltpu.CompilerParams(needs_layout_passes=False),
      name="attention_affine_sc",
  )
  sc_g, sc_l = sc(mask, attention)

  tc = pl.pallas_call(
      _tc_body,
      grid=(TC_B, H // HB),
      in_specs=[
          pl.BlockSpec(memory_space=pltpu.SMEM),
          pl.BlockSpec((1, C, HB, W), lambda b, h: (SC_B + b, 0, h, 0)),
      ],
      out_specs=[
          pl.BlockSpec((1, 1, HB, W), lambda b, h: (b, 0, h, 0)),
          pl.BlockSpec((1, 1, HB, W), lambda b, h: (b, 0, h, 0)),
      ],
      out_shape=(jax.ShapeDtypeStruct((TC_B, 1, H, W), jnp.float32),
                 jax.ShapeDtypeStruct((TC_B, 1, H, W), jnp.float32)),
      name="attention_affine_tc",
  )
  tc_g, tc_l = tc(attention, mask)

  return (lax.dynamic_update_slice(sc_g, tc_g, (SC_B, 0, 0, 0)),
          lax.dynamic_update_slice(sc_l, tc_l, (SC_B, 0, 0, 0)))


def kernel(mask, attention):
  return _run(mask, attention)


# speedup vs baseline: 95.6352x; 1.0004x over previous
import jax
import jax.numpy as jnp
from jax import lax
from jax.experimental import pallas as pl
from jax.experimental.pallas import tpu as pltpu
from jax.experimental.pallas import tpu_sc as plsc

B, C, H, W = 8, 19, 512, 512
NC, NS = 2, 16
NW = NC * NS
SC_B = 4
TC_B = B - SC_B
RB = 8
WC = 256
CH = RB * WC
ROWS_PER_TILE = (SC_B * H) // NW
NWH = W // WC
NCHUNK = (ROWS_PER_TILE // RB) * NWH
NVREG = CH // 16
VPR = WC // 16
HB = 128


def _sc_body(mask_hbm, att_hbm, outg_hbm, outl_hbm,
             att_v, g_v, l_v, in0, in1, og0, og1, ol0, ol1,
             sin0, sin1, sog0, sog1, sol0, sol1):
  wid = lax.axis_index("s") * NC + lax.axis_index("c")
  row_base = wid * ROWS_PER_TILE

  pltpu.sync_copy(att_hbm, att_v)
  for i in (0, 16):
    rows = jnp.minimum(lax.iota(jnp.int32, 16) + i, C - 1)
    a0 = plsc.load_gather(att_v, [rows, jnp.zeros((16,), jnp.int32)])
    a1 = plsc.load_gather(att_v, [rows, jnp.ones((16,), jnp.int32)])
    m = jnp.maximum(a0, a1)
    e0 = jnp.exp(a0 - m)
    e1 = jnp.exp(a1 - m)
    s = e0 + e1
    g_v[pl.ds(i, 16)] = e0 / s
    l_v[pl.ds(i, 16)] = e1 / s

  inbuf = (in0, in1)
  ogbuf = (og0, og1)
  olbuf = (ol0, ol1)
  isem = (sin0, sin1)
  gsem = (sog0, sog1)
  lsem = (sol0, sol1)

  def chunk_addr(k):
    g = row_base + (k // NWH) * RB
    wcol = (k % NWH) * WC
    return g >> 9, pl.multiple_of(g & (H - 1), RB), pl.multiple_of(wcol, WC)

  def start_in(k, p):
    b, r, wcol = chunk_addr(k)
    pltpu.async_copy(
        mask_hbm.at[b, :, pl.ds(r, RB), pl.ds(wcol, WC)], inbuf[p], isem[p])

  def wait_in(p):
    pltpu.make_async_copy(
        mask_hbm.at[0, :, pl.ds(0, RB), pl.ds(0, WC)], inbuf[p], isem[p]).wait()

  def compute(k, p):
    b, r0, wcol = chunk_addr(k)
    buf, og, ol = inbuf[p], ogbuf[p], olbuf[p]

    @plsc.parallel_loop(0, NVREG, 1, unroll=1)
    def vbody(i):
      r = i // VPR
      col = (i % VPR) * 16

      def tree(lo, hi):
        if hi - lo == 1:
          return buf[lo, r, pl.ds(col, 16)], jnp.full((16,), lo, jnp.int32)
        mid = (lo + hi) // 2
        va, ia = tree(lo, mid)
        vb, ib = tree(mid, hi)
        upd = vb > va
        return jnp.where(upd, vb, va), jnp.where(upd, ib, ia)

      _, idx = tree(0, C)
      og[r, pl.ds(col, 16)] = plsc.load_gather(g_v, [idx])
      ol[r, pl.ds(col, 16)] = plsc.load_gather(l_v, [idx])

    pltpu.async_copy(
        og, outg_hbm.at[b, 0, pl.ds(r0, RB), pl.ds(wcol, WC)], gsem[p])
    pltpu.async_copy(
        ol, outl_hbm.at[b, 0, pl.ds(r0, RB), pl.ds(wcol, WC)], lsem[p])

  def wait_out(p):
    pltpu.make_async_copy(
        ogbuf[p], outg_hbm.at[0, 0, pl.ds(0, RB), pl.ds(0, WC)], gsem[p]).wait()
    pltpu.make_async_copy(
        olbuf[p], outl_hbm.at[0, 0, pl.ds(0, RB), pl.ds(0, WC)], lsem[p]).wait()

  start_in(0, 0)

  def pair(j, carry):
    k0 = j * 2
    start_in(k0 + 1, 1)
    wait_in(0)
    lax.cond(j >= 1, lambda: wait_out(0), lambda: None)
    compute(k0, 0)

    lax.cond(j < NCHUNK // 2 - 1, lambda: start_in(k0 + 2, 0), lambda: None)
    wait_in(1)
    lax.cond(j >= 1, lambda: wait_out(1), lambda: None)
    compute(k0 + 1, 1)
    return carry

  lax.fori_loop(0, NCHUNK // 2, pair, 0)
  wait_out(0)
  wait_out(1)


def _tc_body(att_ref, mask_ref, og_ref, ol_ref):
  best = mask_ref[0, 0]
  bd = jnp.full((HB, W), att_ref[0, 0] - att_ref[0, 1], jnp.float32)
  for c in range(1, C):
    x = mask_ref[0, c]
    upd = x > best
    best = jnp.where(upd, x, best)
    bd = jnp.where(upd, att_ref[c, 0] - att_ref[c, 1], bd)
  og_ref[0, 0] = 1.0 / (1.0 + jnp.exp(-bd))
  ol_ref[0, 0] = 1.0 / (1.0 + jnp.exp(bd))


@jax.jit
def _run(mask, attention):
  mesh = plsc.VectorSubcoreMesh(
      core_axis_name="c", subcore_axis_name="s",
      num_cores=NC, num_subcores=NS)
  sc = pl.kernel(
      _sc_body,
      out_type=(jax.ShapeDtypeStruct((B, 1, H, W), jnp.float32),
                jax.ShapeDtypeStruct((B, 1, H, W), jnp.float32)),
      mesh=mesh,
      scratch_types=[
          pltpu.VMEM((C, 2), jnp.float32),
          pltpu.VMEM((32,), jnp.float32),
          pltpu.VMEM((32,), jnp.float32),
          pltpu.VMEM((C, RB, WC), jnp.float32),
          pltpu.VMEM((C, RB, WC), jnp.float32),
          pltpu.VMEM((RB, WC), jnp.float32),
          pltpu.VMEM((RB, WC), jnp.float32),
          pltpu.VMEM((RB, WC), jnp.float32),
          pltpu.VMEM((RB, WC), jnp.float32),
          pltpu.SemaphoreType.DMA,
          pltpu.SemaphoreType.DMA,
          pltpu.SemaphoreType.DMA,
          pltpu.SemaphoreType.DMA,
          pltpu.SemaphoreType.DMA,
          pltpu.SemaphoreType.DMA,
      ],
      compiler_params=pltpu.CompilerParams(needs_layout_passes=False),
      name="attention_affine_sc",
  )
  sc_g, sc_l = sc(mask, attention)

  tc = pl.pallas_call(
      _tc_body,
      grid=(TC_B, H // HB),
      in_specs=[
          pl.BlockSpec(memory_space=pltpu.SMEM),
          pl.BlockSpec((1, C, HB, W), lambda b, h: (SC_B + b, 0, h, 0)),
      ],
      out_specs=[
          pl.BlockSpec((1, 1, HB, W), lambda b, h: (b, 0, h, 0)),
          pl.BlockSpec((1, 1, HB, W), lambda b, h: (b, 0, h, 0)),
      ],
      out_shape=(jax.ShapeDtypeStruct((TC_B, 1, H, W), jnp.float32),
                 jax.ShapeDtypeStruct((TC_B, 1, H, W), jnp.float32)),
      name="attention_affine_tc",
  )
  tc_g, tc_l = tc(attention, mask)

  return (lax.dynamic_update_slice(sc_g, tc_g, (SC_B, 0, 0, 0)),
          lax.dynamic_update_slice(sc_l, tc_l, (SC_B, 0, 0, 0)))


def kernel(mask, attention):
  return _run(mask, attention)
